# baseline (speedup 1.0000x reference)
import jax
import jax.numpy as jnp
from jax import lax
from jax.experimental import pallas as pl
from jax.experimental.pallas import tpu as pltpu
from jax.experimental.pallas import tpu_sc as plsc

NUM_BANDS = 32
B, V, D = 1024, 10475, 3
S, M = 8192, 4096
VW = V * D
OW = (S + NUM_BANDS) * D
OWP = 25600
TOT = B * VW
L = 32768
NC, NS = 2, 16
NW = NC * NS
NB_PER = B // NW
ACCW = 16 * NUM_BANDS * D


def _body(verts_hbm, seg_hbm, bv_hbm, bid_hbm, out_hbm,
          slab, outb, seg_v, bv_v, bid_v, inv_v, acc, dsem, osem):
    wid = lax.axis_index("s") * NC + lax.axis_index("c")

    pltpu.sync_copy(seg_hbm, seg_v)
    pltpu.sync_copy(bv_hbm, bv_v)
    pltpu.sync_copy(bid_hbm, bid_v)

    lanes = lax.iota(jnp.int32, 16)
    lane_base = lanes * (NUM_BANDS * D)
    zeros16 = jnp.zeros((16,), jnp.float32)
    ones16 = jnp.ones((16,), jnp.float32)
    row192 = jnp.full((16,), (S * 3) // 128, jnp.int32)

    def zero_acc():
        def zbody(i, _):
            acc[pl.ds(pl.multiple_of(i * 16, 16), 16)] = zeros16
            return 0
        lax.fori_loop(0, ACCW // 16, zbody, 0)

    zero_acc()

    def cnt_body(j, _):
        bid = bid_v[pl.ds(pl.multiple_of(j * 16, 16), 16)]
        dst = lane_base + bid * 3
        for c in range(3):
            plsc.addupdate_scatter(acc, [dst + c], ones16)
        return 0
    lax.fori_loop(0, M // 16, cnt_body, 0)

    for g in range(NUM_BANDS * D // 16):
        s = zeros16
        for r in range(16):
            s = s + acc[pl.ds(r * (NUM_BANDS * D) + g * 16, 16)]
        inv_v[pl.ds(g * 16, 16)] = 1.0 / jnp.maximum(s, 1.0)

    def batch_body(bi, _):
        b = wid * NB_PER + bi
        start = b * VW
        a0 = jnp.minimum(start - lax.rem(start, 1024), TOT - L)
        r0 = pl.multiple_of(a0 // 128, 8)
        delta = start - a0
        pltpu.sync_copy(verts_hbm.at[pl.ds(r0, L // 128)], slab)
        zero_acc()

        def seg_body(j, _):
            v = seg_v[pl.ds(pl.multiple_of(j * 16, 16), 16)]
            src = v * 3 + delta
            dst = j * 48 + lanes * 3
            for c in range(3):
                a = src + c
                x = plsc.load_gather(slab, [a >> 7, a & 127])
                d = dst + c
                plsc.store_scatter(outb, [d >> 7, d & 127], x)
            return 0
        lax.fori_loop(0, S // 16, seg_body, 0)

        def band_body(j, _):
            bv = bv_v[pl.ds(pl.multiple_of(j * 16, 16), 16)]
            bid = bid_v[pl.ds(pl.multiple_of(j * 16, 16), 16)]
            src = bv * 3 + delta
            dst = lane_base + bid * 3
            for c in range(3):
                a = src + c
                x = plsc.load_gather(slab, [a >> 7, a & 127])
                plsc.addupdate_scatter(acc, [dst + c], x)
            return 0
        lax.fori_loop(0, M // 16, band_body, 0)

        for g in range(NUM_BANDS * D // 16):
            s = zeros16
            for r in range(16):
                s = s + acc[pl.ds(r * (NUM_BANDS * D) + g * 16, 16)]
            val = s * inv_v[pl.ds(g * 16, 16)]
            plsc.store_scatter(outb, [row192, g * 16 + lanes], val)

        pltpu.sync_copy(outb, out_hbm.at[pl.ds(b * (OWP // 128), OWP // 128)])
        return 0
    lax.fori_loop(0, NB_PER, batch_body, 0)


@jax.jit
def kernel(vertices, segment_vidx, band_vidx, band_ids):
    v2 = vertices.reshape(TOT // 128, 128)
    mesh = plsc.VectorSubcoreMesh(core_axis_name="c", subcore_axis_name="s")
    out1 = pl.kernel(
        _body,
        out_type=jax.ShapeDtypeStruct((B * (OWP // 128), 128), jnp.float32),
        mesh=mesh,
        compiler_params=pltpu.CompilerParams(needs_layout_passes=False),
        scratch_types=[
            pltpu.VMEM((L // 128, 128), jnp.float32),
            pltpu.VMEM((OWP // 128, 128), jnp.float32),
            pltpu.VMEM((S,), jnp.int32),
            pltpu.VMEM((M,), jnp.int32),
            pltpu.VMEM((M,), jnp.int32),
            pltpu.VMEM((NUM_BANDS * D,), jnp.float32),
            pltpu.VMEM((ACCW,), jnp.float32),
            pltpu.SemaphoreType.DMA,
            pltpu.SemaphoreType.DMA,
        ],
    )(v2, segment_vidx, band_vidx, band_ids)
    out2 = out1.reshape(B, OWP)[:, :OW]
    return out2.reshape(B, S + NUM_BANDS, D)


# speedup vs baseline: 19.6337x; 19.6337x over previous
import jax
import jax.numpy as jnp
from jax import lax
from jax.experimental import pallas as pl
from jax.experimental.pallas import tpu as pltpu
from jax.experimental.pallas import tpu_sc as plsc

NUM_BANDS = 32
B, V, D = 1024, 10475, 3
S, M = 8192, 4096
VW = V * D
OW = (S + NUM_BANDS) * D
NC, NS = 2, 16
NW = NC * NS
NB_PER = B // NW
ACCW = 16 * NUM_BANDS * D


def _body(verts_hbm, seg_hbm, bv_hbm, bid_hbm, out_hbm,
          slab, outb, seg_v, bv_v, bid_v, inv_v, acc, dsem, osem):
    wid = lax.axis_index("s") * NC + lax.axis_index("c")

    pltpu.sync_copy(seg_hbm, seg_v)
    pltpu.sync_copy(bv_hbm, bv_v)
    pltpu.sync_copy(bid_hbm, bid_v)

    lanes = lax.iota(jnp.int32, 16)
    lane_base = lanes * (NUM_BANDS * D)
    zeros16 = jnp.zeros((16,), jnp.float32)
    ones16 = jnp.ones((16,), jnp.float32)

    def zero_acc():
        def zbody(i, _):
            acc[pl.ds(pl.multiple_of(i * 16, 16), 16)] = zeros16
            return 0
        lax.fori_loop(0, ACCW // 16, zbody, 0)

    zero_acc()

    def cnt_body(j, _):
        bid = bid_v[pl.ds(pl.multiple_of(j * 16, 16), 16)]
        dst = lane_base + bid * 3
        for c in range(3):
            plsc.addupdate_scatter(acc, [dst + c], ones16)
        return 0
    lax.fori_loop(0, M // 16, cnt_body, 0)

    for g in range(NUM_BANDS * D // 16):
        s = zeros16
        for r in range(16):
            s = s + acc[pl.ds(r * (NUM_BANDS * D) + g * 16, 16)]
        inv_v[pl.ds(g * 16, 16)] = 1.0 / jnp.maximum(s, 1.0)

    def slab_copy(bi):
        b = wid * NB_PER + bi
        return pltpu.make_async_copy(verts_hbm.at[b], slab.at[lax.rem(bi, 2)],
                                     dsem)

    slab_copy(0).start()

    def batch_body(bi, _):
        slab_copy(bi).wait()
        slot = lax.rem(bi, 2)
        slot16 = jnp.broadcast_to(slot, (16,))

        @pl.when(bi + 1 < NB_PER)
        def _():
            slab_copy(bi + 1).start()

        b = wid * NB_PER + bi
        zero_acc()

        def seg_body(j, _):
            v = seg_v[pl.ds(pl.multiple_of(j * 16, 16), 16)]
            src = v * 3
            dst = j * 48 + lanes * 3
            for c in range(3):
                x = plsc.load_gather(slab, [slot16, src + c])
                plsc.store_scatter(outb, [dst + c], x)
            return 0
        lax.fori_loop(0, S // 16, seg_body, 0)

        def band_body(j, _):
            bv = bv_v[pl.ds(pl.multiple_of(j * 16, 16), 16)]
            bid = bid_v[pl.ds(pl.multiple_of(j * 16, 16), 16)]
            src = bv * 3
            dst = lane_base + bid * 3
            for c in range(3):
                x = plsc.load_gather(slab, [slot16, src + c])
                plsc.addupdate_scatter(acc, [dst + c], x)
            return 0
        lax.fori_loop(0, M // 16, band_body, 0)

        for g in range(NUM_BANDS * D // 16):
            s = zeros16
            for r in range(16):
                s = s + acc[pl.ds(r * (NUM_BANDS * D) + g * 16, 16)]
            outb[pl.ds(S * 3 + g * 16, 16)] = s * inv_v[pl.ds(g * 16, 16)]

        pltpu.sync_copy(outb, out_hbm.at[b])
        return 0
    lax.fori_loop(0, NB_PER, batch_body, 0)


@jax.jit
def kernel(vertices, segment_vidx, band_vidx, band_ids):
    verts2 = vertices.reshape(B, VW)
    mesh = plsc.VectorSubcoreMesh(core_axis_name="c", subcore_axis_name="s")
    out2 = pl.kernel(
        _body,
        out_type=jax.ShapeDtypeStruct((B, OW), jnp.float32),
        mesh=mesh,
        compiler_params=pltpu.CompilerParams(needs_layout_passes=False),
        scratch_types=[
            pltpu.VMEM((2, VW), jnp.float32),
            pltpu.VMEM((OW,), jnp.float32),
            pltpu.VMEM((S,), jnp.int32),
            pltpu.VMEM((M,), jnp.int32),
            pltpu.VMEM((M,), jnp.int32),
            pltpu.VMEM((NUM_BANDS * D,), jnp.float32),
            pltpu.VMEM((ACCW,), jnp.float32),
            pltpu.SemaphoreType.DMA,
            pltpu.SemaphoreType.DMA,
        ],
    )(verts2, segment_vidx, band_vidx, band_ids)
    return out2.reshape(B, S + NUM_BANDS, D)


# speedup vs baseline: 19.7833x; 1.0076x over previous
import jax
import jax.numpy as jnp
from jax import lax
from jax.experimental import pallas as pl
from jax.experimental.pallas import tpu as pltpu
from jax.experimental.pallas import tpu_sc as plsc

NUM_BANDS = 32
B, V, D = 1024, 10475, 3
S, M = 8192, 4096
VW = V * D
OW = (S + NUM_BANDS) * D
NC, NS = 2, 16
NW = NC * NS
NB_PER = B // NW
ACCW = 16 * NUM_BANDS * D


def _body(verts_hbm, seg_hbm, bv_hbm, bid_hbm, out_hbm,
          slab, outb, seg_v, bv_v, bid_v, inv_v, acc, dsem, osem):
    wid = lax.axis_index("s") * NC + lax.axis_index("c")

    pltpu.sync_copy(seg_hbm, seg_v)
    pltpu.sync_copy(bv_hbm, bv_v)
    pltpu.sync_copy(bid_hbm, bid_v)

    lanes = lax.iota(jnp.int32, 16)
    lane_base = lanes * (NUM_BANDS * D)
    zeros16 = jnp.zeros((16,), jnp.float32)
    ones16 = jnp.ones((16,), jnp.float32)

    def zero_acc():
        def zbody(i, _):
            acc[pl.ds(pl.multiple_of(i * 16, 16), 16)] = zeros16
            return 0
        lax.fori_loop(0, ACCW // 16, zbody, 0)

    zero_acc()

    def cnt_body(j, _):
        bid = bid_v[pl.ds(pl.multiple_of(j * 16, 16), 16)]
        dst = lane_base + bid * 3
        for c in range(3):
            plsc.addupdate_scatter(acc, [dst + c], ones16)
        return 0
    lax.fori_loop(0, M // 16, cnt_body, 0)

    for g in range(NUM_BANDS * D // 16):
        s = zeros16
        for r in range(16):
            s = s + acc[pl.ds(r * (NUM_BANDS * D) + g * 16, 16)]
        inv_v[pl.ds(g * 16, 16)] = 1.0 / jnp.maximum(s, 1.0)

    def batch_body(bi, _):
        b = wid * NB_PER + bi
        pltpu.sync_copy(verts_hbm.at[b], slab)
        zero_acc()

        def seg_body(j, _):
            v = seg_v[pl.ds(pl.multiple_of(j * 16, 16), 16)]
            src = v * 3
            dst = j * 48 + lanes * 3
            for c in range(3):
                x = plsc.load_gather(slab, [src + c])
                plsc.store_scatter(outb, [dst + c], x)
            return 0
        lax.fori_loop(0, S // 16, seg_body, 0)

        def band_body(j, _):
            bv = bv_v[pl.ds(pl.multiple_of(j * 16, 16), 16)]
            bid = bid_v[pl.ds(pl.multiple_of(j * 16, 16), 16)]
            src = bv * 3
            dst = lane_base + bid * 3
            for c in range(3):
                x = plsc.load_gather(slab, [src + c])
                plsc.addupdate_scatter(acc, [dst + c], x)
            return 0
        lax.fori_loop(0, M // 16, band_body, 0)

        for g in range(NUM_BANDS * D // 16):
            s = zeros16
            for r in range(16):
                s = s + acc[pl.ds(r * (NUM_BANDS * D) + g * 16, 16)]
            outb[pl.ds(S * 3 + g * 16, 16)] = s * inv_v[pl.ds(g * 16, 16)]

        pltpu.sync_copy(outb, out_hbm.at[b])
        return 0
    lax.fori_loop(0, NB_PER, batch_body, 0)


@jax.jit
def kernel(vertices, segment_vidx, band_vidx, band_ids):
    verts2 = vertices.reshape(B, VW)
    mesh = plsc.VectorSubcoreMesh(core_axis_name="c", subcore_axis_name="s")
    out2 = pl.kernel(
        _body,
        out_type=jax.ShapeDtypeStruct((B, OW), jnp.float32),
        mesh=mesh,
        compiler_params=pltpu.CompilerParams(needs_layout_passes=False),
        scratch_types=[
            pltpu.VMEM((VW,), jnp.float32),
            pltpu.VMEM((OW,), jnp.float32),
            pltpu.VMEM((S,), jnp.int32),
            pltpu.VMEM((M,), jnp.int32),
            pltpu.VMEM((M,), jnp.int32),
            pltpu.VMEM((NUM_BANDS * D,), jnp.float32),
            pltpu.VMEM((ACCW,), jnp.float32),
            pltpu.SemaphoreType.DMA,
            pltpu.SemaphoreType.DMA,
        ],
    )(verts2, segment_vidx, band_vidx, band_ids)
    return out2.reshape(B, S + NUM_BANDS, D)
